# baseline (speedup 1.0000x reference)
import functools

import jax
import jax.numpy as jnp
from jax import lax
from jax.experimental import pallas as pl
from jax.experimental.pallas import tpu as pltpu
from jax.experimental.pallas import tpu_sc as plsc

_B = 16384
_D = 64
_N = 1000000
_NC, _NS = 2, 16
_NW = _NC * _NS
_RPW = _B // _NW
_G = 16
_NG = _RPW // _G


def _tr_body(src, dst):
    dst[...] = src[...].T


_TCOLS = 512


def _to_row_major(tab):
    return pl.pallas_call(
        _tr_body,
        grid=(pl.cdiv(_N, _TCOLS),),
        in_specs=[pl.BlockSpec((_D, _TCOLS), lambda i: (0, i))],
        out_specs=pl.BlockSpec((_TCOLS, _D), lambda i: (i, 0)),
        out_shape=jax.ShapeDtypeStruct((_N, _D), jnp.float32),
    )(tab.T)


def _gather_body(idx_hbm, tab, out, idx_v, buf, gsem, wsem):
    wid = lax.axis_index("s") * _NC + lax.axis_index("c")
    base = wid * _RPW
    pltpu.sync_copy(idx_hbm.at[pl.ds(base, _RPW)], idx_v)

    def fire(j, _):
        iv = idx_v[pl.ds(j * _G, _G)]
        pltpu.async_copy(tab.at[iv], buf.at[pl.ds(j * _G, _G)], gsem)
        return 0

    lax.fori_loop(0, _NG, fire, 0)

    def drain(j, _):
        pltpu.make_async_copy(tab.at[pl.ds(0, _G)], buf.at[pl.ds(0, _G)],
                              gsem).wait()
        return 0

    lax.fori_loop(0, _NG, drain, 0)
    pltpu.async_copy(buf, out.at[pl.ds(base, _RPW)], wsem).wait()


_gather1 = functools.partial(
    pl.kernel,
    out_type=jax.ShapeDtypeStruct((_B, _D), jnp.float32),
    mesh=plsc.VectorSubcoreMesh(core_axis_name="c", subcore_axis_name="s"),
    scratch_types=[
        pltpu.VMEM((_RPW,), jnp.int32),
        pltpu.VMEM((_RPW, _D), jnp.float32),
        pltpu.SemaphoreType.DMA,
        pltpu.SemaphoreType.DMA,
    ],
    compiler_params=pltpu.CompilerParams(use_tc_tiling_on_sc=False),
)(_gather_body)


_BLK = 2048


def _mlp_body(xmfu_ref, xmfi_ref, xu_ref, xi_ref,
              w1_ref, b1_ref, w2_ref, b2_ref, w3_ref, b3_ref,
              wf_ref, bf_ref, out_ref):
    dn = (((1,), (1,)), ((), ()))
    f32 = jnp.float32
    w1 = w1_ref[...]
    h = lax.dot_general(xu_ref[...], w1[:, :_D], dn, preferred_element_type=f32)
    h = h + lax.dot_general(xi_ref[...], w1[:, _D:], dn, preferred_element_type=f32)
    h = jnp.maximum(h + b1_ref[...], 0.0)
    h = lax.dot_general(h, w2_ref[...], dn, preferred_element_type=f32)
    h = jnp.maximum(h + b2_ref[...], 0.0)
    h = lax.dot_general(h, w3_ref[...], dn, preferred_element_type=f32)
    h = jnp.maximum(h + b3_ref[...], 0.0)
    xmf = xmfu_ref[...] * xmfi_ref[...]
    wf = wf_ref[...]
    logit = lax.dot_general(xmf, wf[:, :_D], dn, preferred_element_type=f32)
    logit = logit + lax.dot_general(h, wf[:, _D:], dn, preferred_element_type=f32)
    out_ref[...] = jax.nn.sigmoid(logit + bf_ref[...])


def kernel(user, item, mf_user_embed, mf_item_embed, mlp_user_embed,
           mlp_item_embed, W1, b1, W2, b2, W3, b3, Wf, bf):
    xmfu = _gather1(user, _to_row_major(mf_user_embed))
    xmfi = _gather1(item, _to_row_major(mf_item_embed))
    xu = _gather1(user, _to_row_major(mlp_user_embed))
    xi = _gather1(item, _to_row_major(mlp_item_embed))
    full = lambda shape: pl.BlockSpec(shape, lambda i: (0,) * len(shape))
    row = lambda w: pl.BlockSpec((_BLK, w), lambda i: (i, 0))
    out = pl.pallas_call(
        _mlp_body,
        grid=(_B // _BLK,),
        in_specs=[
            row(_D), row(_D), row(_D), row(_D),
            full((64, 128)), full((1, 64)),
            full((32, 64)), full((1, 32)),
            full((16, 32)), full((1, 16)),
            full((1, 80)), full((1, 1)),
        ],
        out_specs=pl.BlockSpec((_BLK, 1), lambda i: (i, 0)),
        out_shape=jax.ShapeDtypeStruct((_B, 1), jnp.float32),
    )(xmfu, xmfi, xu, xi,
      W1, b1.reshape(1, 64), W2, b2.reshape(1, 32), W3, b3.reshape(1, 16),
      Wf, bf.reshape(1, 1))
    return out


# speedup vs baseline: 3.7033x; 3.7033x over previous
import functools

import jax
import jax.numpy as jnp
from jax import lax
from jax.experimental import pallas as pl
from jax.experimental.pallas import tpu as pltpu
from jax.experimental.pallas import tpu_sc as plsc

_B = 16384
_D = 64
_W = 2 * _D
_NC, _NS = 2, 16
_NW = _NC * _NS
_RPW = _B // _NW
_G = 16
_NG = _RPW // _G


def _gather_body(idx_hbm, tab, out, idx_v, buf, gsem, wsem):
    wid = lax.axis_index("s") * _NC + lax.axis_index("c")
    base = wid * _RPW
    pltpu.sync_copy(idx_hbm.at[pl.ds(base, _RPW)], idx_v)

    def fire(j, _):
        iv = idx_v[pl.ds(j * _G, _G)]
        pltpu.async_copy(tab.at[iv], buf.at[pl.ds(j * _G, _G)], gsem)
        return 0

    lax.fori_loop(0, _NG, fire, 0)

    def drain(j, _):
        pltpu.make_async_copy(tab.at[pl.ds(0, _G)], buf.at[pl.ds(0, _G)],
                              gsem).wait()
        return 0

    lax.fori_loop(0, _NG, drain, 0)
    pltpu.async_copy(buf, out.at[pl.ds(base, _RPW)], wsem).wait()


_gather2 = functools.partial(
    pl.kernel,
    out_type=jax.ShapeDtypeStruct((_B, _W), jnp.float32),
    mesh=plsc.VectorSubcoreMesh(core_axis_name="c", subcore_axis_name="s"),
    scratch_types=[
        pltpu.VMEM((_RPW,), jnp.int32),
        pltpu.VMEM((_RPW, _W), jnp.float32),
        pltpu.SemaphoreType.DMA,
        pltpu.SemaphoreType.DMA,
    ],
    compiler_params=pltpu.CompilerParams(use_tc_tiling_on_sc=False),
)(_gather_body)


_BLK = 2048


def _mlp_body(gu_ref, gi_ref,
              w1_ref, b1_ref, w2_ref, b2_ref, w3_ref, b3_ref,
              wf_ref, bf_ref, out_ref):
    dn = (((1,), (1,)), ((), ()))
    f32 = jnp.float32
    gu = gu_ref[...]
    gi = gi_ref[...]
    w1 = w1_ref[...]
    h = lax.dot_general(gu[:, _D:], w1[:, :_D], dn, preferred_element_type=f32)
    h = h + lax.dot_general(gi[:, _D:], w1[:, _D:], dn, preferred_element_type=f32)
    h = jnp.maximum(h + b1_ref[...], 0.0)
    h = lax.dot_general(h, w2_ref[...], dn, preferred_element_type=f32)
    h = jnp.maximum(h + b2_ref[...], 0.0)
    h = lax.dot_general(h, w3_ref[...], dn, preferred_element_type=f32)
    h = jnp.maximum(h + b3_ref[...], 0.0)
    xmf = gu[:, :_D] * gi[:, :_D]
    wf = wf_ref[...]
    logit = lax.dot_general(xmf, wf[:, :_D], dn, preferred_element_type=f32)
    logit = logit + lax.dot_general(h, wf[:, _D:], dn, preferred_element_type=f32)
    out_ref[...] = jax.nn.sigmoid(logit + bf_ref[...])


def kernel(user, item, mf_user_embed, mf_item_embed, mlp_user_embed,
           mlp_item_embed, W1, b1, W2, b2, W3, b3, Wf, bf):
    cu = jnp.concatenate([mf_user_embed, mlp_user_embed], axis=1)
    ci = jnp.concatenate([mf_item_embed, mlp_item_embed], axis=1)
    gu = _gather2(user, cu)
    gi = _gather2(item, ci)
    full = lambda shape: pl.BlockSpec(shape, lambda i: (0,) * len(shape))
    row = lambda w: pl.BlockSpec((_BLK, w), lambda i: (i, 0))
    out = pl.pallas_call(
        _mlp_body,
        grid=(_B // _BLK,),
        in_specs=[
            row(_W), row(_W),
            full((64, 128)), full((1, 64)),
            full((32, 64)), full((1, 32)),
            full((16, 32)), full((1, 16)),
            full((1, 80)), full((1, 1)),
        ],
        out_specs=pl.BlockSpec((_BLK, 1), lambda i: (i, 0)),
        out_shape=jax.ShapeDtypeStruct((_B, 1), jnp.float32),
    )(gu, gi,
      W1, b1.reshape(1, 64), W2, b2.reshape(1, 32), W3, b3.reshape(1, 16),
      Wf, bf.reshape(1, 1))
    return out
